# Initial kernel scaffold; baseline (speedup 1.0000x reference)
import jax
import jax.numpy as jnp
from jax.experimental import pallas as pl


def kernel(x, edge, W, a):
    raise NotImplementedError("write your pallas kernel here")


# speedup vs baseline: 23.0375x; 23.0375x over previous
import functools

import jax
import jax.numpy as jnp
from jax import lax
from jax.experimental import pallas as pl
from jax.experimental.pallas import tpu as pltpu
from jax.experimental.pallas import tpu_sc as plsc

N = 10000
E = 320000
IN_F = 128
ATT_DIM = 128
HEADS = 8
D_K = ATT_DIM // HEADS
ALPHA = 0.2

N_PAD = 10240
CH = 128
NCHUNK = E // CH
BN = 1000


def _tc_body(x_ref, w_ref, af_ref, ag_ref, wx_ref, f_ref, g_ref, m_ref, msc):
    i = pl.program_id(0)
    wx = jnp.dot(x_ref[...], w_ref[...], preferred_element_type=jnp.float32,
                 precision=lax.Precision.HIGHEST)
    wx_ref[...] = wx
    f = jnp.dot(wx, af_ref[...], preferred_element_type=jnp.float32,
                precision=lax.Precision.HIGHEST)
    g = jnp.dot(wx, ag_ref[...], preferred_element_type=jnp.float32,
                precision=lax.Precision.HIGHEST)
    f_ref[...] = f
    g_ref[...] = g
    bf = jnp.max(f)
    bg = jnp.max(g)

    @pl.when(i == 0)
    def _():
        msc[0] = bf
        msc[1] = bg

    @pl.when(i > 0)
    def _():
        msc[0] = jnp.maximum(msc[0], bf)
        msc[1] = jnp.maximum(msc[1], bg)

    s = msc[0] + msc[1]
    m_ref[...] = jnp.broadcast_to(jnp.maximum(s, ALPHA * s), (1, 1))


def _tc_call(x, W, af, ag):
    return pl.pallas_call(
        _tc_body,
        grid=(N // BN,),
        in_specs=[
            pl.BlockSpec((BN, IN_F), lambda i: (i, 0)),
            pl.BlockSpec((IN_F, ATT_DIM), lambda i: (0, 0)),
            pl.BlockSpec((ATT_DIM, HEADS), lambda i: (0, 0)),
            pl.BlockSpec((ATT_DIM, HEADS), lambda i: (0, 0)),
        ],
        out_specs=[
            pl.BlockSpec((BN, ATT_DIM), lambda i: (i, 0)),
            pl.BlockSpec((BN, HEADS), lambda i: (i, 0)),
            pl.BlockSpec((BN, HEADS), lambda i: (i, 0)),
            pl.BlockSpec((1, 1), lambda i: (0, 0)),
        ],
        out_shape=[
            jax.ShapeDtypeStruct((N, ATT_DIM), jnp.float32),
            jax.ShapeDtypeStruct((N, HEADS), jnp.float32),
            jax.ShapeDtypeStruct((N, HEADS), jnp.float32),
            jax.ShapeDtypeStruct((1, 1), jnp.float32),
        ],
        scratch_shapes=[pltpu.SMEM((2,), jnp.float32)],
    )(x, W, af, ag)


@functools.lru_cache(maxsize=None)
def _sc_kernels(nc, ns):
    nw = nc * ns
    iters = -(-NCHUNK // nw)
    rows = N_PAD // ns
    mesh = plsc.VectorSubcoreMesh(core_axis_name="c", subcore_axis_name="s",
                                  num_cores=nc, num_subcores=ns)

    def k1(e0, e1, ftab, gtab, mvec_h, zeros_h, ex_h, p0_h, p1_h,
           idx0, idx1, fs, gd, ex, mv, denom, sem0, sem1):
        cid = lax.axis_index("c")
        sid = lax.axis_index("s")
        wid = sid * nc + cid
        pltpu.sync_copy(zeros_h.at[pl.ds(sid * rows, rows)],
                        denom.at[pl.ds(sid * rows, rows)])
        pltpu.sync_copy(mvec_h, mv)
        plsc.subcore_barrier()
        mreg = mv[...]
        iot = lax.iota(jnp.int32, 16)
        rdiv = iot >> 3
        cmod = iot & 7

        def chunk(i, carry):
            c = wid + i * nw

            @pl.when(c < NCHUNK)
            def _():
                base = c * CH
                pltpu.sync_copy(e0.at[pl.ds(base, CH)], idx0)
                pltpu.sync_copy(e1.at[pl.ds(base, CH)], idx1)
                cp0 = pltpu.async_copy(ftab.at[idx0], fs, sem0)
                cp1 = pltpu.async_copy(gtab.at[idx1], gd, sem1)
                cp0.wait()
                cp1.wait()

                def step(j, carry2):
                    r = rdiv + 2 * j
                    s = (plsc.load_gather(fs, [r, cmod])
                         + plsc.load_gather(gd, [r, cmod]))
                    s = jnp.where(s >= 0, s, ALPHA * s)
                    plsc.store_scatter(ex, [r, cmod], jnp.exp(s - mreg))
                    return carry2

                lax.fori_loop(0, CH // 2, step, 0)
                pltpu.sync_copy(ex, ex_h.at[pl.ds(base, CH)])
                pltpu.sync_copy(ex, denom.at[idx0], add=True)

            return carry

        lax.fori_loop(0, iters, chunk, 0)
        plsc.subcore_barrier()

        @pl.when(cid == 0)
        def _():
            pltpu.sync_copy(denom.at[pl.ds(sid * rows, rows)],
                            p0_h.at[pl.ds(sid * rows, rows)])

        @pl.when(cid == 1)
        def _():
            pltpu.sync_copy(denom.at[pl.ds(sid * rows, rows)],
                            p1_h.at[pl.ds(sid * rows, rows)])

    scp = pltpu.CompilerParams(needs_layout_passes=False, use_tc_tiling_on_sc=False)
    k1c = pl.kernel(
        k1,
        compiler_params=scp,
        out_type=(
            jax.ShapeDtypeStruct((E, HEADS), jnp.float32),
            jax.ShapeDtypeStruct((N_PAD, HEADS), jnp.float32),
            jax.ShapeDtypeStruct((N_PAD, HEADS), jnp.float32),
        ),
        mesh=mesh,
        scratch_types=[
            pltpu.VMEM((CH,), jnp.int32),
            pltpu.VMEM((CH,), jnp.int32),
            pltpu.VMEM((CH, HEADS), jnp.float32),
            pltpu.VMEM((CH, HEADS), jnp.float32),
            pltpu.VMEM((CH, HEADS), jnp.float32),
            pltpu.VMEM((16,), jnp.float32),
            pltpu.VMEM_SHARED((N_PAD, HEADS), jnp.float32),
            pltpu.SemaphoreType.DMA,
            pltpu.SemaphoreType.DMA,
        ],
    )

    def k2(e0, ex_h, p0_h, p1_h, att_h, idx0, exb, d0, d1, att, sem0, sem1):
        cid = lax.axis_index("c")
        sid = lax.axis_index("s")
        wid = sid * nc + cid
        iot = lax.iota(jnp.int32, 16)
        rdiv = iot >> 3
        cmod = iot & 7

        def chunk(i, carry):
            c = wid + i * nw

            @pl.when(c < NCHUNK)
            def _():
                base = c * CH
                pltpu.sync_copy(e0.at[pl.ds(base, CH)], idx0)
                pltpu.sync_copy(ex_h.at[pl.ds(base, CH)], exb)
                cp0 = pltpu.async_copy(p0_h.at[idx0], d0, sem0)
                cp1 = pltpu.async_copy(p1_h.at[idx0], d1, sem1)
                cp0.wait()
                cp1.wait()

                def step(j, carry2):
                    r = rdiv + 2 * j
                    dv = (plsc.load_gather(d0, [r, cmod])
                          + plsc.load_gather(d1, [r, cmod]) + 1e-16)
                    ev = plsc.load_gather(exb, [r, cmod])
                    plsc.store_scatter(att, [r, cmod], ev / dv)
                    return carry2

                lax.fori_loop(0, CH // 2, step, 0)
                pltpu.sync_copy(att, att_h.at[pl.ds(base, CH)])

            return carry

        lax.fori_loop(0, iters, chunk, 0)

    k2c = pl.kernel(
        k2,
        compiler_params=scp,
        out_type=jax.ShapeDtypeStruct((E, HEADS), jnp.float32),
        mesh=mesh,
        scratch_types=[
            pltpu.VMEM((CH,), jnp.int32),
            pltpu.VMEM((CH, HEADS), jnp.float32),
            pltpu.VMEM((CH, HEADS), jnp.float32),
            pltpu.VMEM((CH, HEADS), jnp.float32),
            pltpu.VMEM((CH, HEADS), jnp.float32),
            pltpu.SemaphoreType.DMA,
            pltpu.SemaphoreType.DMA,
        ],
    )
    return k1c, k2c


def kernel(x, edge, W, a):
    av = a.reshape(2 * D_K)
    blk = (jnp.arange(ATT_DIM)[:, None] // D_K
           == jnp.arange(HEADS)[None, :]).astype(jnp.float32)
    af = blk * jnp.tile(av[:D_K], HEADS)[:, None]
    ag = blk * jnp.tile(av[D_K:], HEADS)[:, None]
    wx, f, g, m = _tc_call(x, W, af, ag)
    mvec = jnp.broadcast_to(m[0, 0], (16,))

    try:
        info = plsc.get_sparse_core_info()
        nc, ns = info.num_cores, info.num_subcores
    except Exception:
        nc, ns = 2, 16
    k1c, k2c = _sc_kernels(nc, ns)

    e0 = edge[0, 0]
    e1 = edge[0, 1]
    zeros = jnp.zeros((N_PAD, HEADS), jnp.float32)
    ex, p0, p1 = k1c(e0, e1, f, g, mvec, zeros)
    att = k2c(e0, ex, p0, p1)
    return att, wx


# speedup vs baseline: 31.4635x; 1.3657x over previous
import functools

import jax
import jax.numpy as jnp
from jax import lax
from jax.experimental import pallas as pl
from jax.experimental.pallas import tpu as pltpu
from jax.experimental.pallas import tpu_sc as plsc

N = 10000
E = 320000
IN_F = 128
ATT_DIM = 128
HEADS = 8
D_K = ATT_DIM // HEADS
ALPHA = 0.2

N_PAD = 10240
CH = 512
NCHUNK = E // CH
BN = 1000


def _tc_body(x_ref, w_ref, af_ref, ag_ref, wx_ref, f_ref, g_ref, m_ref, msc):
    i = pl.program_id(0)
    wx = jnp.dot(x_ref[...], w_ref[...], preferred_element_type=jnp.float32,
                 precision=lax.Precision.HIGHEST)
    wx_ref[...] = wx
    f = jnp.dot(wx, af_ref[...], preferred_element_type=jnp.float32,
                precision=lax.Precision.HIGHEST)
    g = jnp.dot(wx, ag_ref[...], preferred_element_type=jnp.float32,
                precision=lax.Precision.HIGHEST)
    f_ref[...] = f
    g_ref[...] = g
    bf = jnp.max(f)
    bg = jnp.max(g)

    @pl.when(i == 0)
    def _():
        msc[0] = bf
        msc[1] = bg

    @pl.when(i > 0)
    def _():
        msc[0] = jnp.maximum(msc[0], bf)
        msc[1] = jnp.maximum(msc[1], bg)

    s = msc[0] + msc[1]
    m_ref[...] = jnp.broadcast_to(jnp.maximum(s, ALPHA * s), (1, 1))


def _tc_call(x, W, af, ag):
    return pl.pallas_call(
        _tc_body,
        grid=(N // BN,),
        in_specs=[
            pl.BlockSpec((BN, IN_F), lambda i: (i, 0)),
            pl.BlockSpec((IN_F, ATT_DIM), lambda i: (0, 0)),
            pl.BlockSpec((ATT_DIM, HEADS), lambda i: (0, 0)),
            pl.BlockSpec((ATT_DIM, HEADS), lambda i: (0, 0)),
        ],
        out_specs=[
            pl.BlockSpec((BN, ATT_DIM), lambda i: (i, 0)),
            pl.BlockSpec((BN, HEADS), lambda i: (i, 0)),
            pl.BlockSpec((BN, HEADS), lambda i: (i, 0)),
            pl.BlockSpec((1, 1), lambda i: (0, 0)),
        ],
        out_shape=[
            jax.ShapeDtypeStruct((N, ATT_DIM), jnp.float32),
            jax.ShapeDtypeStruct((N, HEADS), jnp.float32),
            jax.ShapeDtypeStruct((N, HEADS), jnp.float32),
            jax.ShapeDtypeStruct((1, 1), jnp.float32),
        ],
        scratch_shapes=[pltpu.SMEM((2,), jnp.float32)],
    )(x, W, af, ag)


@functools.lru_cache(maxsize=None)
def _sc_kernels(nc, ns):
    nw = nc * ns
    iters = -(-NCHUNK // nw)
    rows = N_PAD // ns
    mesh = plsc.VectorSubcoreMesh(core_axis_name="c", subcore_axis_name="s",
                                  num_cores=nc, num_subcores=ns)

    def k1(e0, e1, ftab, gtab, mvec_h, zeros_h, ex_h, p0_h, p1_h,
           idx0, idx1, fs, gd, ex, mv, denom, sem0, sem1):
        cid = lax.axis_index("c")
        sid = lax.axis_index("s")
        wid = sid * nc + cid
        pltpu.sync_copy(zeros_h.at[pl.ds(sid * rows, rows)],
                        denom.at[pl.ds(sid * rows, rows)])
        pltpu.sync_copy(mvec_h, mv)
        plsc.subcore_barrier()
        mreg = mv[...]
        iot = lax.iota(jnp.int32, 16)
        rdiv = iot >> 3
        cmod = iot & 7

        def chunk(i, carry):
            c = wid + i * nw

            @pl.when(c < NCHUNK)
            def _():
                base = c * CH
                pltpu.sync_copy(e0.at[pl.ds(base, CH)], idx0)
                pltpu.sync_copy(e1.at[pl.ds(base, CH)], idx1)
                cp0 = pltpu.async_copy(ftab.at[idx0], fs, sem0)
                cp1 = pltpu.async_copy(gtab.at[idx1], gd, sem1)
                cp0.wait()
                cp1.wait()

                def step(j, carry2):
                    r = rdiv + 2 * j
                    s = (plsc.load_gather(fs, [r, cmod])
                         + plsc.load_gather(gd, [r, cmod]))
                    s = jnp.where(s >= 0, s, ALPHA * s)
                    plsc.store_scatter(ex, [r, cmod], jnp.exp(s - mreg))
                    return carry2

                lax.fori_loop(0, CH // 2, step, 0)
                pltpu.sync_copy(ex, ex_h.at[pl.ds(base, CH)])
                pltpu.sync_copy(ex, denom.at[idx0], add=True)

            return carry

        lax.fori_loop(0, iters, chunk, 0)
        plsc.subcore_barrier()

        @pl.when(cid == 0)
        def _():
            pltpu.sync_copy(denom.at[pl.ds(sid * rows, rows)],
                            p0_h.at[pl.ds(sid * rows, rows)])

        @pl.when(cid == 1)
        def _():
            pltpu.sync_copy(denom.at[pl.ds(sid * rows, rows)],
                            p1_h.at[pl.ds(sid * rows, rows)])

    scp = pltpu.CompilerParams(needs_layout_passes=False, use_tc_tiling_on_sc=False)
    k1c = pl.kernel(
        k1,
        compiler_params=scp,
        out_type=(
            jax.ShapeDtypeStruct((E, HEADS), jnp.float32),
            jax.ShapeDtypeStruct((N_PAD, HEADS), jnp.float32),
            jax.ShapeDtypeStruct((N_PAD, HEADS), jnp.float32),
        ),
        mesh=mesh,
        scratch_types=[
            pltpu.VMEM((CH,), jnp.int32),
            pltpu.VMEM((CH,), jnp.int32),
            pltpu.VMEM((CH, HEADS), jnp.float32),
            pltpu.VMEM((CH, HEADS), jnp.float32),
            pltpu.VMEM((CH, HEADS), jnp.float32),
            pltpu.VMEM((16,), jnp.float32),
            pltpu.VMEM_SHARED((N_PAD, HEADS), jnp.float32),
            pltpu.SemaphoreType.DMA,
            pltpu.SemaphoreType.DMA,
        ],
    )

    def k2(e0, ex_h, p0_h, p1_h, att_h, idx0, exb, d0, d1, att, sem0, sem1):
        cid = lax.axis_index("c")
        sid = lax.axis_index("s")
        wid = sid * nc + cid
        iot = lax.iota(jnp.int32, 16)
        rdiv = iot >> 3
        cmod = iot & 7

        def chunk(i, carry):
            c = wid + i * nw

            @pl.when(c < NCHUNK)
            def _():
                base = c * CH
                pltpu.sync_copy(e0.at[pl.ds(base, CH)], idx0)
                pltpu.sync_copy(ex_h.at[pl.ds(base, CH)], exb)
                cp0 = pltpu.async_copy(p0_h.at[idx0], d0, sem0)
                cp1 = pltpu.async_copy(p1_h.at[idx0], d1, sem1)
                cp0.wait()
                cp1.wait()

                def step(j, carry2):
                    r = rdiv + 2 * j
                    dv = (plsc.load_gather(d0, [r, cmod])
                          + plsc.load_gather(d1, [r, cmod]) + 1e-16)
                    ev = plsc.load_gather(exb, [r, cmod])
                    plsc.store_scatter(att, [r, cmod], ev / dv)
                    return carry2

                lax.fori_loop(0, CH // 2, step, 0)
                pltpu.sync_copy(att, att_h.at[pl.ds(base, CH)])

            return carry

        lax.fori_loop(0, iters, chunk, 0)

    k2c = pl.kernel(
        k2,
        compiler_params=scp,
        out_type=jax.ShapeDtypeStruct((E, HEADS), jnp.float32),
        mesh=mesh,
        scratch_types=[
            pltpu.VMEM((CH,), jnp.int32),
            pltpu.VMEM((CH, HEADS), jnp.float32),
            pltpu.VMEM((CH, HEADS), jnp.float32),
            pltpu.VMEM((CH, HEADS), jnp.float32),
            pltpu.VMEM((CH, HEADS), jnp.float32),
            pltpu.SemaphoreType.DMA,
            pltpu.SemaphoreType.DMA,
        ],
    )
    return k1c, k2c


def kernel(x, edge, W, a):
    av = a.reshape(2 * D_K)
    blk = (jnp.arange(ATT_DIM)[:, None] // D_K
           == jnp.arange(HEADS)[None, :]).astype(jnp.float32)
    af = blk * jnp.tile(av[:D_K], HEADS)[:, None]
    ag = blk * jnp.tile(av[D_K:], HEADS)[:, None]
    wx, f, g, m = _tc_call(x, W, af, ag)
    mvec = jnp.broadcast_to(m[0, 0], (16,))

    try:
        info = plsc.get_sparse_core_info()
        nc, ns = info.num_cores, info.num_subcores
    except Exception:
        nc, ns = 2, 16
    k1c, k2c = _sc_kernels(nc, ns)

    e0 = edge[0, 0]
    e1 = edge[0, 1]
    zeros = jnp.zeros((N_PAD, HEADS), jnp.float32)
    ex, p0, p1 = k1c(e0, e1, f, g, mvec, zeros)
    att = k2c(e0, ex, p0, p1)
    return att, wx


# speedup vs baseline: 41.0754x; 1.3055x over previous
import functools

import jax
import jax.numpy as jnp
from jax import lax
from jax.experimental import pallas as pl
from jax.experimental.pallas import tpu as pltpu
from jax.experimental.pallas import tpu_sc as plsc

N = 10000
E = 320000
IN_F = 128
ATT_DIM = 128
HEADS = 8
D_K = ATT_DIM // HEADS
ALPHA = 0.2

N_PAD = 10240
CH = 512
NCHUNK = E // CH
BN = 1000


def _tc_body(x_ref, w_ref, af_ref, ag_ref, wx_ref, f_ref, g_ref, m_ref, msc):
    i = pl.program_id(0)
    wx = jnp.dot(x_ref[...], w_ref[...], preferred_element_type=jnp.float32,
                 precision=lax.Precision.HIGHEST)
    wx_ref[...] = wx
    f = jnp.dot(wx, af_ref[...], preferred_element_type=jnp.float32,
                precision=lax.Precision.HIGHEST)
    g = jnp.dot(wx, ag_ref[...], preferred_element_type=jnp.float32,
                precision=lax.Precision.HIGHEST)
    f_ref[...] = f
    g_ref[...] = g
    bf = jnp.max(f)
    bg = jnp.max(g)

    @pl.when(i == 0)
    def _():
        msc[0] = bf
        msc[1] = bg

    @pl.when(i > 0)
    def _():
        msc[0] = jnp.maximum(msc[0], bf)
        msc[1] = jnp.maximum(msc[1], bg)

    s = msc[0] + msc[1]
    m_ref[...] = jnp.broadcast_to(jnp.maximum(s, ALPHA * s), (1, 1))


def _tc_call(x, W, af, ag):
    return pl.pallas_call(
        _tc_body,
        grid=(N // BN,),
        in_specs=[
            pl.BlockSpec((BN, IN_F), lambda i: (i, 0)),
            pl.BlockSpec((IN_F, ATT_DIM), lambda i: (0, 0)),
            pl.BlockSpec((ATT_DIM, HEADS), lambda i: (0, 0)),
            pl.BlockSpec((ATT_DIM, HEADS), lambda i: (0, 0)),
        ],
        out_specs=[
            pl.BlockSpec((BN, ATT_DIM), lambda i: (i, 0)),
            pl.BlockSpec((BN, HEADS), lambda i: (i, 0)),
            pl.BlockSpec((BN, HEADS), lambda i: (i, 0)),
            pl.BlockSpec((1, 1), lambda i: (0, 0)),
        ],
        out_shape=[
            jax.ShapeDtypeStruct((N, ATT_DIM), jnp.float32),
            jax.ShapeDtypeStruct((N, HEADS), jnp.float32),
            jax.ShapeDtypeStruct((N, HEADS), jnp.float32),
            jax.ShapeDtypeStruct((1, 1), jnp.float32),
        ],
        scratch_shapes=[pltpu.SMEM((2,), jnp.float32)],
    )(x, W, af, ag)


@functools.lru_cache(maxsize=None)
def _sc_kernels(nc, ns):
    nw = nc * ns
    iters = -(-NCHUNK // nw)
    rows = N_PAD // ns
    mesh = plsc.VectorSubcoreMesh(core_axis_name="c", subcore_axis_name="s",
                                  num_cores=nc, num_subcores=ns)

    def k1(edge3, ftab, gtab, mvec_h, zeros_h, ex_h, p0_h, p1_h,
           ia0, ia1, fsA, fsB, gdA, gdB, exA, exB, mv, denom,
           semp, sgA, sgB, swA, swB, ssA, ssB):
        cid = lax.axis_index("c")
        sid = lax.axis_index("s")
        wid = sid * nc + cid
        pltpu.sync_copy(zeros_h.at[pl.ds(sid * rows, rows)],
                        denom.at[pl.ds(sid * rows, rows)])
        pltpu.sync_copy(mvec_h, mv)
        plsc.subcore_barrier()
        mreg = mv[...]
        fsl, gdl, exl = [fsA, fsB], [gdA, gdB], [exA, exB]
        sgl, swl, ssl = [sgA, sgB], [swA, swB], [ssA, ssB]

        for j in range(iters):
            c = wid + j * nw

            @pl.when(c < NCHUNK)
            def _(c=c, j=j):
                pltpu.async_copy(edge3.at[0, c], ia0.at[j], semp)
                pltpu.async_copy(edge3.at[1, c], ia1.at[j], semp)

        for j in range(iters):
            c = wid + j * nw

            @pl.when(c < NCHUNK)
            def _(c=c, j=j):
                pltpu.make_async_copy(edge3.at[0, c], ia0.at[j], semp).wait()
                pltpu.make_async_copy(edge3.at[1, c], ia1.at[j], semp).wait()

        def issue_gath(j):
            c = wid + j * nw
            s = j & 1

            @pl.when(c < NCHUNK)
            def _():
                pltpu.async_copy(ftab.at[ia0.at[j]], fsl[s], sgl[s])
                pltpu.async_copy(gtab.at[ia1.at[j]], gdl[s], sgl[s])

        def wait_gath(j):
            c = wid + j * nw
            s = j & 1

            @pl.when(c < NCHUNK)
            def _():
                pltpu.make_async_copy(ftab.at[ia0.at[j]], fsl[s], sgl[s]).wait()
                pltpu.make_async_copy(gtab.at[ia1.at[j]], gdl[s], sgl[s]).wait()

        def wait_wb(j):
            if j < 0:
                return
            c = wid + j * nw
            s = j & 1

            @pl.when(c < NCHUNK)
            def _():
                pltpu.make_async_copy(exl[s], ex_h.at[pl.ds(c * CH, CH)], swl[s]).wait()
                pltpu.make_async_copy(exl[s], denom.at[ia0.at[j]], ssl[s]).wait()

        iot = lax.iota(jnp.int32, 16)
        rdiv = iot >> 3
        cmod = iot & 7

        issue_gath(0)
        issue_gath(1)
        for j in range(iters):
            c = wid + j * nw
            s = j & 1
            wait_gath(j)
            wait_wb(j - 2)

            @pl.when(c < NCHUNK)
            def _(c=c, s=s, j=j):
                def step(jj, carry):
                    r = rdiv + 2 * jj
                    v = (plsc.load_gather(fsl[s], [r, cmod])
                         + plsc.load_gather(gdl[s], [r, cmod]))
                    v = jnp.where(v >= 0, v, ALPHA * v)
                    plsc.store_scatter(exl[s], [r, cmod], jnp.exp(v - mreg))
                    return carry

                lax.fori_loop(0, CH // 2, step, 0)
                pltpu.async_copy(exl[s], ex_h.at[pl.ds(c * CH, CH)], swl[s])
                pltpu.async_copy(exl[s], denom.at[ia0.at[j]], ssl[s], add=True)

            issue_gath(j + 2)
        wait_wb(iters - 2)
        wait_wb(iters - 1)

        plsc.subcore_barrier()

        @pl.when(cid == 0)
        def _():
            pltpu.sync_copy(denom.at[pl.ds(sid * rows, rows)],
                            p0_h.at[pl.ds(sid * rows, rows)])

        @pl.when(cid == 1)
        def _():
            pltpu.sync_copy(denom.at[pl.ds(sid * rows, rows)],
                            p1_h.at[pl.ds(sid * rows, rows)])

    scp = pltpu.CompilerParams(needs_layout_passes=False, use_tc_tiling_on_sc=False)
    k1c = pl.kernel(
        k1,
        compiler_params=scp,
        out_type=(
            jax.ShapeDtypeStruct((E, HEADS), jnp.float32),
            jax.ShapeDtypeStruct((N_PAD, HEADS), jnp.float32),
            jax.ShapeDtypeStruct((N_PAD, HEADS), jnp.float32),
        ),
        mesh=mesh,
        scratch_types=[
            pltpu.VMEM((iters, CH), jnp.int32),
            pltpu.VMEM((iters, CH), jnp.int32),
            pltpu.VMEM((CH, HEADS), jnp.float32),
            pltpu.VMEM((CH, HEADS), jnp.float32),
            pltpu.VMEM((CH, HEADS), jnp.float32),
            pltpu.VMEM((CH, HEADS), jnp.float32),
            pltpu.VMEM((CH, HEADS), jnp.float32),
            pltpu.VMEM((CH, HEADS), jnp.float32),
            pltpu.VMEM((16,), jnp.float32),
            pltpu.VMEM_SHARED((N_PAD, HEADS), jnp.float32),
            pltpu.SemaphoreType.DMA,
            pltpu.SemaphoreType.DMA,
            pltpu.SemaphoreType.DMA,
            pltpu.SemaphoreType.DMA,
            pltpu.SemaphoreType.DMA,
            pltpu.SemaphoreType.DMA,
            pltpu.SemaphoreType.DMA,
        ],
    )

    def k2(edge3, ex_h, p0_h, p1_h, att_h,
           ia0, exbA, exbB, d0A, d0B, d1A, d1B, atA, atB,
           semp, sgA, sgB, swA, swB):
        cid = lax.axis_index("c")
        sid = lax.axis_index("s")
        wid = sid * nc + cid
        exbl, d0l, d1l, atl = [exbA, exbB], [d0A, d0B], [d1A, d1B], [atA, atB]
        sgl, swl = [sgA, sgB], [swA, swB]

        for j in range(iters):
            c = wid + j * nw

            @pl.when(c < NCHUNK)
            def _(c=c, j=j):
                pltpu.async_copy(edge3.at[0, c], ia0.at[j], semp)

        for j in range(iters):
            c = wid + j * nw

            @pl.when(c < NCHUNK)
            def _(c=c, j=j):
                pltpu.make_async_copy(edge3.at[0, c], ia0.at[j], semp).wait()

        def issue_gath(j):
            c = wid + j * nw
            s = j & 1

            @pl.when(c < NCHUNK)
            def _():
                pltpu.async_copy(ex_h.at[pl.ds(c * CH, CH)], exbl[s], sgl[s])
                pltpu.async_copy(p0_h.at[ia0.at[j]], d0l[s], sgl[s])
                pltpu.async_copy(p1_h.at[ia0.at[j]], d1l[s], sgl[s])

        def wait_gath(j):
            c = wid + j * nw
            s = j & 1

            @pl.when(c < NCHUNK)
            def _():
                pltpu.make_async_copy(ex_h.at[pl.ds(c * CH, CH)], exbl[s], sgl[s]).wait()
                pltpu.make_async_copy(p0_h.at[ia0.at[j]], d0l[s], sgl[s]).wait()
                pltpu.make_async_copy(p1_h.at[ia0.at[j]], d1l[s], sgl[s]).wait()

        def wait_wb(j):
            if j < 0:
                return
            c = wid + j * nw
            s = j & 1

            @pl.when(c < NCHUNK)
            def _():
                pltpu.make_async_copy(atl[s], att_h.at[pl.ds(c * CH, CH)], swl[s]).wait()

        iot = lax.iota(jnp.int32, 16)
        rdiv = iot >> 3
        cmod = iot & 7

        issue_gath(0)
        issue_gath(1)
        for j in range(iters):
            c = wid + j * nw
            s = j & 1
            wait_gath(j)
            wait_wb(j - 2)

            @pl.when(c < NCHUNK)
            def _(c=c, s=s):
                def step(jj, carry):
                    r = rdiv + 2 * jj
                    dv = (plsc.load_gather(d0l[s], [r, cmod])
                          + plsc.load_gather(d1l[s], [r, cmod]) + 1e-16)
                    ev = plsc.load_gather(exbl[s], [r, cmod])
                    plsc.store_scatter(atl[s], [r, cmod], ev / dv)
                    return carry

                lax.fori_loop(0, CH // 2, step, 0)
                pltpu.async_copy(atl[s], att_h.at[pl.ds(c * CH, CH)], swl[s])

            issue_gath(j + 2)
        wait_wb(iters - 2)
        wait_wb(iters - 1)

    k2c = pl.kernel(
        k2,
        compiler_params=scp,
        out_type=jax.ShapeDtypeStruct((E, HEADS), jnp.float32),
        mesh=mesh,
        scratch_types=[
            pltpu.VMEM((iters, CH), jnp.int32),
            pltpu.VMEM((CH, HEADS), jnp.float32),
            pltpu.VMEM((CH, HEADS), jnp.float32),
            pltpu.VMEM((CH, HEADS), jnp.float32),
            pltpu.VMEM((CH, HEADS), jnp.float32),
            pltpu.VMEM((CH, HEADS), jnp.float32),
            pltpu.VMEM((CH, HEADS), jnp.float32),
            pltpu.VMEM((CH, HEADS), jnp.float32),
            pltpu.VMEM((CH, HEADS), jnp.float32),
            pltpu.SemaphoreType.DMA,
            pltpu.SemaphoreType.DMA,
            pltpu.SemaphoreType.DMA,
            pltpu.SemaphoreType.DMA,
            pltpu.SemaphoreType.DMA,
        ],
    )
    return k1c, k2c


def kernel(x, edge, W, a):
    av = a.reshape(2 * D_K)
    blk = (jnp.arange(ATT_DIM)[:, None] // D_K
           == jnp.arange(HEADS)[None, :]).astype(jnp.float32)
    af = blk * jnp.tile(av[:D_K], HEADS)[:, None]
    ag = blk * jnp.tile(av[D_K:], HEADS)[:, None]
    wx, f, g, m = _tc_call(x, W, af, ag)
    mvec = jnp.broadcast_to(m[0, 0], (16,))

    try:
        info = plsc.get_sparse_core_info()
        nc, ns = info.num_cores, info.num_subcores
    except Exception:
        nc, ns = 2, 16
    k1c, k2c = _sc_kernels(nc, ns)

    edge3 = edge.reshape(2, NCHUNK, CH)
    zeros = jnp.zeros((N_PAD, HEADS), jnp.float32)
    ex, p0, p1 = k1c(edge3, f, g, mvec, zeros)
    att = k2c(edge3, ex, p0, p1)
    return att, wx


# speedup vs baseline: 68.2220x; 1.6609x over previous
import functools

import jax
import jax.numpy as jnp
from jax import lax
from jax.experimental import pallas as pl
from jax.experimental.pallas import tpu as pltpu
from jax.experimental.pallas import tpu_sc as plsc

N = 10000
E = 320000
IN_F = 128
ATT_DIM = 128
HEADS = 8
D_K = ATT_DIM // HEADS
ALPHA = 0.2

N_PAD = 10240
CH = 512
NCHUNK = E // CH
BN = 1000


def _tc_body(x_ref, w_ref, af_ref, ag_ref, wx_ref, fg_ref, m_ref, msc):
    i = pl.program_id(0)
    wx = jnp.dot(x_ref[...], w_ref[...], preferred_element_type=jnp.float32)
    wx_ref[...] = wx
    f = jnp.dot(wx, af_ref[...], preferred_element_type=jnp.float32,
                precision=lax.Precision.HIGHEST)
    g = jnp.dot(wx, ag_ref[...], preferred_element_type=jnp.float32,
                precision=lax.Precision.HIGHEST)
    fg_ref[...] = jnp.concatenate([f, g], axis=1)
    bf = jnp.max(f)
    bg = jnp.max(g)

    @pl.when(i == 0)
    def _():
        msc[0] = bf
        msc[1] = bg

    @pl.when(i > 0)
    def _():
        msc[0] = jnp.maximum(msc[0], bf)
        msc[1] = jnp.maximum(msc[1], bg)

    s = msc[0] + msc[1]
    m_ref[...] = jnp.broadcast_to(jnp.maximum(s, ALPHA * s), (1, 1))


def _tc_call(x, W, af, ag):
    return pl.pallas_call(
        _tc_body,
        grid=(N // BN,),
        in_specs=[
            pl.BlockSpec((BN, IN_F), lambda i: (i, 0)),
            pl.BlockSpec((IN_F, ATT_DIM), lambda i: (0, 0)),
            pl.BlockSpec((ATT_DIM, HEADS), lambda i: (0, 0)),
            pl.BlockSpec((ATT_DIM, HEADS), lambda i: (0, 0)),
        ],
        out_specs=[
            pl.BlockSpec((BN, ATT_DIM), lambda i: (i, 0)),
            pl.BlockSpec((BN, 2 * HEADS), lambda i: (i, 0)),
            pl.BlockSpec((1, 1), lambda i: (0, 0)),
        ],
        out_shape=[
            jax.ShapeDtypeStruct((N, ATT_DIM), jnp.float32),
            jax.ShapeDtypeStruct((N, 2 * HEADS), jnp.float32),
            jax.ShapeDtypeStruct((1, 1), jnp.float32),
        ],
        scratch_shapes=[pltpu.SMEM((2,), jnp.float32)],
    )(x, W, af, ag)


@functools.lru_cache(maxsize=None)
def _sc_kernels(nc, ns):
    nw = nc * ns
    iters = -(-NCHUNK // nw)
    rows = N_PAD // ns
    mesh = plsc.VectorSubcoreMesh(core_axis_name="c", subcore_axis_name="s",
                                  num_cores=nc, num_subcores=ns)

    def k1(edge3, fgtab, mvec_h, zeros_h, ex_h, p0_h, p1_h,
           ia0, ia1, fsA, fsB, gdA, gdB, exA, exB, mv, denom,
           semp, sgA, sgB, swA, swB, ssA, ssB):
        cid = lax.axis_index("c")
        sid = lax.axis_index("s")
        wid = sid * nc + cid
        pltpu.sync_copy(zeros_h.at[pl.ds(sid * rows, rows)],
                        denom.at[pl.ds(sid * rows, rows)])
        pltpu.sync_copy(mvec_h, mv)
        plsc.subcore_barrier()
        mreg = mv[...]
        fsl, gdl, exl = [fsA, fsB], [gdA, gdB], [exA, exB]
        sgl, swl, ssl = [sgA, sgB], [swA, swB], [ssA, ssB]

        for j in range(iters):
            c = wid + j * nw

            @pl.when(c < NCHUNK)
            def _(c=c, j=j):
                pltpu.async_copy(edge3.at[0, c], ia0.at[j], semp)
                pltpu.async_copy(edge3.at[1, c], ia1.at[j], semp)

        for j in range(iters):
            c = wid + j * nw

            @pl.when(c < NCHUNK)
            def _(c=c, j=j):
                pltpu.make_async_copy(edge3.at[0, c], ia0.at[j], semp).wait()
                pltpu.make_async_copy(edge3.at[1, c], ia1.at[j], semp).wait()

        def issue_gath(j):
            c = wid + j * nw
            s = j & 1

            @pl.when(c < NCHUNK)
            def _():
                pltpu.async_copy(fgtab.at[ia0.at[j]], fsl[s], sgl[s])
                pltpu.async_copy(fgtab.at[ia1.at[j]], gdl[s], sgl[s])

        def wait_gath(j):
            c = wid + j * nw
            s = j & 1

            @pl.when(c < NCHUNK)
            def _():
                pltpu.make_async_copy(fgtab.at[ia0.at[j]], fsl[s], sgl[s]).wait()
                pltpu.make_async_copy(fgtab.at[ia1.at[j]], gdl[s], sgl[s]).wait()

        def wait_wb(j):
            if j < 0:
                return
            c = wid + j * nw
            s = j & 1

            @pl.when(c < NCHUNK)
            def _():
                pltpu.make_async_copy(exl[s], ex_h.at[pl.ds(c * CH, CH)], swl[s]).wait()
                pltpu.make_async_copy(exl[s], denom.at[ia0.at[j]], ssl[s]).wait()

        iot = lax.iota(jnp.int32, 16)
        rdiv = iot >> 3
        cmod = iot & 7
        cmod8 = cmod + 8

        issue_gath(0)
        issue_gath(1)
        for j in range(iters):
            c = wid + j * nw
            s = j & 1
            wait_gath(j)
            wait_wb(j - 2)

            @pl.when(c < NCHUNK)
            def _(c=c, s=s, j=j):
                def step(jj, carry):
                    r = rdiv + 2 * jj
                    v = (plsc.load_gather(fsl[s], [r, cmod])
                         + plsc.load_gather(gdl[s], [r, cmod8]))
                    v = jnp.where(v >= 0, v, ALPHA * v)
                    plsc.store_scatter(exl[s], [r, cmod], jnp.exp(v - mreg))
                    return carry

                lax.fori_loop(0, CH // 2, step, 0)
                pltpu.async_copy(exl[s], ex_h.at[pl.ds(c * CH, CH)], swl[s])
                pltpu.async_copy(exl[s], denom.at[ia0.at[j]], ssl[s], add=True)

            issue_gath(j + 2)
        wait_wb(iters - 2)
        wait_wb(iters - 1)

        plsc.subcore_barrier()

        @pl.when(cid == 0)
        def _():
            pltpu.sync_copy(denom.at[pl.ds(sid * rows, rows)],
                            p0_h.at[pl.ds(sid * rows, rows)])

        @pl.when(cid == 1)
        def _():
            pltpu.sync_copy(denom.at[pl.ds(sid * rows, rows)],
                            p1_h.at[pl.ds(sid * rows, rows)])

    scp = pltpu.CompilerParams(needs_layout_passes=False, use_tc_tiling_on_sc=False)
    k1c = pl.kernel(
        k1,
        compiler_params=scp,
        out_type=(
            jax.ShapeDtypeStruct((E, HEADS), jnp.float32),
            jax.ShapeDtypeStruct((N_PAD, HEADS), jnp.float32),
            jax.ShapeDtypeStruct((N_PAD, HEADS), jnp.float32),
        ),
        mesh=mesh,
        scratch_types=[
            pltpu.VMEM((iters, CH), jnp.int32),
            pltpu.VMEM((iters, CH), jnp.int32),
            pltpu.VMEM((CH, 2 * HEADS), jnp.float32),
            pltpu.VMEM((CH, 2 * HEADS), jnp.float32),
            pltpu.VMEM((CH, 2 * HEADS), jnp.float32),
            pltpu.VMEM((CH, 2 * HEADS), jnp.float32),
            pltpu.VMEM((CH, HEADS), jnp.float32),
            pltpu.VMEM((CH, HEADS), jnp.float32),
            pltpu.VMEM((16,), jnp.float32),
            pltpu.VMEM_SHARED((N_PAD, HEADS), jnp.float32),
            pltpu.SemaphoreType.DMA,
            pltpu.SemaphoreType.DMA,
            pltpu.SemaphoreType.DMA,
            pltpu.SemaphoreType.DMA,
            pltpu.SemaphoreType.DMA,
            pltpu.SemaphoreType.DMA,
            pltpu.SemaphoreType.DMA,
        ],
    )

    def k2(edge3, ex_h, p0_h, p1_h, att_h,
           ia0, exbA, exbB, d0A, d0B, d1A, d1B, atA, atB,
           semp, sgA, sgB, swA, swB):
        cid = lax.axis_index("c")
        sid = lax.axis_index("s")
        wid = sid * nc + cid
        exbl, d0l, d1l, atl = [exbA, exbB], [d0A, d0B], [d1A, d1B], [atA, atB]
        sgl, swl = [sgA, sgB], [swA, swB]

        for j in range(iters):
            c = wid + j * nw

            @pl.when(c < NCHUNK)
            def _(c=c, j=j):
                pltpu.async_copy(edge3.at[0, c], ia0.at[j], semp)

        for j in range(iters):
            c = wid + j * nw

            @pl.when(c < NCHUNK)
            def _(c=c, j=j):
                pltpu.make_async_copy(edge3.at[0, c], ia0.at[j], semp).wait()

        def issue_gath(j):
            c = wid + j * nw
            s = j & 1

            @pl.when(c < NCHUNK)
            def _():
                pltpu.async_copy(ex_h.at[pl.ds(c * CH, CH)], exbl[s], sgl[s])
                pltpu.async_copy(p0_h.at[ia0.at[j]], d0l[s], sgl[s])
                pltpu.async_copy(p1_h.at[ia0.at[j]], d1l[s], sgl[s])

        def wait_gath(j):
            c = wid + j * nw
            s = j & 1

            @pl.when(c < NCHUNK)
            def _():
                pltpu.make_async_copy(ex_h.at[pl.ds(c * CH, CH)], exbl[s], sgl[s]).wait()
                pltpu.make_async_copy(p0_h.at[ia0.at[j]], d0l[s], sgl[s]).wait()
                pltpu.make_async_copy(p1_h.at[ia0.at[j]], d1l[s], sgl[s]).wait()

        def wait_wb(j):
            if j < 0:
                return
            c = wid + j * nw
            s = j & 1

            @pl.when(c < NCHUNK)
            def _():
                pltpu.make_async_copy(atl[s], att_h.at[pl.ds(c * (CH // 128), CH // 128)],
                                      swl[s]).wait()

        iot = lax.iota(jnp.int32, 16)
        rdiv = iot >> 3
        cmod = iot & 7

        issue_gath(0)
        issue_gath(1)
        for j in range(iters):
            c = wid + j * nw
            s = j & 1
            wait_gath(j)
            wait_wb(j - 2)

            @pl.when(c < NCHUNK)
            def _(c=c, s=s):
                for blk in range(CH // 128):
                    blkv = jnp.full((16,), blk, jnp.int32)

                    def step(jj, carry, blk=blk, blkv=blkv):
                        m = rdiv + 2 * jj
                        r = m + blk * 128
                        dv = (plsc.load_gather(d0l[s], [r, cmod])
                              + plsc.load_gather(d1l[s], [r, cmod]) + 1e-16)
                        ev = plsc.load_gather(exbl[s], [r, cmod])
                        plsc.store_scatter(atl[s], [blkv, cmod, m], ev / dv)
                        return carry

                    lax.fori_loop(0, 64, step, 0)
                pltpu.async_copy(atl[s], att_h.at[pl.ds(c * (CH // 128), CH // 128)], swl[s])

            issue_gath(j + 2)
        wait_wb(iters - 2)
        wait_wb(iters - 1)

    k2c = pl.kernel(
        k2,
        compiler_params=scp,
        out_type=jax.ShapeDtypeStruct((E // 128, HEADS, 128), jnp.float32),
        mesh=mesh,
        scratch_types=[
            pltpu.VMEM((iters, CH), jnp.int32),
            pltpu.VMEM((CH, HEADS), jnp.float32),
            pltpu.VMEM((CH, HEADS), jnp.float32),
            pltpu.VMEM((CH, HEADS), jnp.float32),
            pltpu.VMEM((CH, HEADS), jnp.float32),
            pltpu.VMEM((CH, HEADS), jnp.float32),
            pltpu.VMEM((CH, HEADS), jnp.float32),
            pltpu.VMEM((CH // 128, HEADS, 128), jnp.float32),
            pltpu.VMEM((CH // 128, HEADS, 128), jnp.float32),
            pltpu.SemaphoreType.DMA,
            pltpu.SemaphoreType.DMA,
            pltpu.SemaphoreType.DMA,
            pltpu.SemaphoreType.DMA,
            pltpu.SemaphoreType.DMA,
        ],
    )
    return k1c, k2c


def kernel(x, edge, W, a):
    av = a.reshape(2 * D_K)
    blk = (jnp.arange(ATT_DIM)[:, None] // D_K
           == jnp.arange(HEADS)[None, :]).astype(jnp.float32)
    af = blk * jnp.tile(av[:D_K], HEADS)[:, None]
    ag = blk * jnp.tile(av[D_K:], HEADS)[:, None]
    wx, fg, m = _tc_call(x, W, af, ag)
    mvec = jnp.broadcast_to(m[0, 0], (16,))

    try:
        info = plsc.get_sparse_core_info()
        nc, ns = info.num_cores, info.num_subcores
    except Exception:
        nc, ns = 2, 16
    k1c, k2c = _sc_kernels(nc, ns)

    edge3 = edge.reshape(2, NCHUNK, CH)
    zeros = jnp.zeros((N_PAD, HEADS), jnp.float32)
    ex, p0, p1 = k1c(edge3, fg, mvec, zeros)
    att3 = k2c(edge3, ex, p0, p1)
    att = att3.transpose(0, 2, 1).reshape(E, HEADS)
    return att, wx
